# Initial kernel scaffold; baseline (speedup 1.0000x reference)
import jax
import jax.numpy as jnp
from jax import lax
from jax.experimental import pallas as pl
from jax.experimental.pallas import tpu as pltpu
from jax.experimental.pallas import tpu_sc as plsc

B = 4096
L = 200
D = 64
NC = 2
NS = 16
NW = NC * NS
ROWS = B * L
RPW = ROWS // NW
CHUNK = 128
NCHUNK = RPW // CHUNK
LANES = 16
VPR = D // LANES


def _sc_body(seq_hbm, pos2_hbm, item_hbm, out_hbm, posv, idxv, rowsv, gsem):
    cid = lax.axis_index("c")
    sid = lax.axis_index("s")
    wid = sid * NC + cid
    base = wid * RPW

    pltpu.sync_copy(pos2_hbm, posv)

    @pl.loop(0, NCHUNK)
    def chunk_loop(c):
        off = c * CHUNK
        l0 = lax.rem(off, L)
        pltpu.sync_copy(seq_hbm.at[pl.ds(base + off, CHUNK)], idxv)
        pltpu.async_copy(item_hbm.at[idxv], rowsv, gsem).wait()

        @pl.loop(0, CHUNK)
        def row_loop(j):
            for k in range(VPR):
                sl = pl.ds(k * LANES, LANES)
                rowsv[j, sl] = rowsv[j, sl] + posv[l0 + j, sl]

        pltpu.sync_copy(rowsv, out_hbm.at[pl.ds(base + off, CHUNK)])


@jax.jit
def _sc_call(seq_flat, pos2, item_embedding):
    mesh = plsc.VectorSubcoreMesh(
        core_axis_name="c", subcore_axis_name="s", num_cores=NC, num_subcores=NS
    )
    return pl.kernel(
        _sc_body,
        out_type=jax.ShapeDtypeStruct((ROWS, D), jnp.float32),
        mesh=mesh,
        scratch_types=[
            pltpu.VMEM((L + CHUNK, D), jnp.float32),
            pltpu.VMEM((CHUNK,), jnp.int32),
            pltpu.VMEM((CHUNK, D), jnp.float32),
            pltpu.SemaphoreType.DMA,
        ],
    )(seq_flat, pos2, item_embedding)


def kernel(seq, pos, neg, item_embedding, position_embedding):
    del pos, neg
    seq_flat = seq.reshape(-1).astype(jnp.int32)
    pos2 = jnp.concatenate(
        [position_embedding, position_embedding[:CHUNK]], axis=0
    )
    out = _sc_call(seq_flat, pos2, item_embedding)
    return out.reshape(B, L, D)


# speedup vs baseline: 1.7706x; 1.7706x over previous
import jax
import jax.numpy as jnp
from jax import lax
from jax.experimental import pallas as pl
from jax.experimental.pallas import tpu as pltpu
from jax.experimental.pallas import tpu_sc as plsc

B = 4096
L = 200
D = 64
NC = 2
NS = 16
NW = NC * NS
ROWS = B * L
RPW = ROWS // NW
CHUNK = 128
NCHUNK = RPW // CHUNK
LANES = 16
VPR = D // LANES


def _sc_body(seq_hbm, pos2_hbm, item_hbm, out_hbm, posv, idxv, rowsv, gsem):
    cid = lax.axis_index("c")
    sid = lax.axis_index("s")
    wid = sid * NC + cid
    base = wid * RPW

    pltpu.sync_copy(pos2_hbm, posv)

    @pl.loop(0, NCHUNK)
    def chunk_loop(c):
        off = c * CHUNK
        l0 = lax.rem(off, L)
        pltpu.sync_copy(seq_hbm.at[pl.ds(base + off, CHUNK)], idxv)
        pltpu.async_copy(item_hbm.at[idxv], rowsv, gsem).wait()

        @pl.loop(0, CHUNK)
        def row_loop(j):
            for k in range(VPR):
                sl = pl.ds(k * LANES, LANES)
                rowsv[j, sl] = rowsv[j, sl] + posv[l0 + j, sl]

        pltpu.sync_copy(rowsv, out_hbm.at[pl.ds(base + off, CHUNK)])


@jax.jit
def _sc_call(seq_flat, pos2, item_embedding):
    mesh = plsc.VectorSubcoreMesh(
        core_axis_name="c", subcore_axis_name="s", num_cores=NC, num_subcores=NS
    )
    return pl.kernel(
        _sc_body,
        out_type=jax.ShapeDtypeStruct((ROWS, D), jnp.float32),
        mesh=mesh,
        compiler_params=pltpu.CompilerParams(use_tc_tiling_on_sc=False),
        scratch_types=[
            pltpu.VMEM((L + CHUNK, D), jnp.float32),
            pltpu.VMEM((CHUNK,), jnp.int32),
            pltpu.VMEM((CHUNK, D), jnp.float32),
            pltpu.SemaphoreType.DMA,
        ],
    )(seq_flat, pos2, item_embedding)


def kernel(seq, pos, neg, item_embedding, position_embedding):
    del pos, neg
    seq_flat = seq.reshape(-1).astype(jnp.int32)
    pos2 = jnp.concatenate(
        [position_embedding, position_embedding[:CHUNK]], axis=0
    )
    out = _sc_call(seq_flat, pos2, item_embedding)
    return out.reshape(B, L, D)


# speedup vs baseline: 2.1604x; 1.2201x over previous
import jax
import jax.numpy as jnp
from jax import lax
from jax.experimental import pallas as pl
from jax.experimental.pallas import tpu as pltpu
from jax.experimental.pallas import tpu_sc as plsc

B = 4096
L = 200
D = 64
NC = 2
NS = 16
NW = NC * NS
ROWS = B * L
RPW = ROWS // NW
CHUNK = 128
NCHUNK = RPW // CHUNK
NBUF = 4
LANES = 16
VPR = D // LANES


def _sc_body(seq_hbm, pos2_hbm, item_hbm, out_hbm, posv, idxv, rowsv, *sems):
    gsems = sems[:NBUF]
    osems = sems[NBUF:]
    cid = lax.axis_index("c")
    sid = lax.axis_index("s")
    wid = sid * NC + cid

    pltpu.sync_copy(seq_hbm.at[pl.ds(wid * NCHUNK, NCHUNK)], idxv)
    pltpu.sync_copy(pos2_hbm, posv)

    def gather_start(c, b):
        pltpu.async_copy(
            item_hbm.at[idxv.at[c]],
            rowsv.at[pl.ds(b * CHUNK, CHUNK)],
            gsems[b],
        )

    def out_copy(c, b, start):
        desc = pltpu.make_async_copy(
            rowsv.at[pl.ds(b * CHUNK, CHUNK)],
            out_hbm.at[pl.ds((wid * NCHUNK + c) * CHUNK, CHUNK)],
            osems[b],
        )
        if start:
            desc.start()
        else:
            desc.wait()

    for b in range(NBUF - 1):
        gather_start(b, b)

    @pl.loop(0, NCHUNK, step=NBUF)
    def outer(C):
        for b in range(NBUF):
            c = C + b
            bg = (b + NBUF - 1) % NBUF
            g = c + NBUF - 1

            pltpu.make_async_copy(
                item_hbm.at[idxv.at[c]],
                rowsv.at[pl.ds(b * CHUNK, CHUNK)],
                gsems[b],
            ).wait()

            l0 = lax.rem(c * CHUNK, L)
            base = b * CHUNK

            @pl.loop(0, CHUNK)
            def row_loop(j):
                for k in range(VPR):
                    sl = pl.ds(k * LANES, LANES)
                    rowsv[base + j, sl] = rowsv[base + j, sl] + posv[l0 + j, sl]

            out_copy(c, b, start=True)

            @pl.when(g < NCHUNK)
            def _():
                @pl.when(c >= 1)
                def _():
                    out_copy(c - 1, bg, start=False)

                gather_start(g, bg)

    for b in range(NBUF):
        c = NCHUNK - NBUF + b
        out_copy(c, b, start=False)


@jax.jit
def _sc_call(seq2, pos2, item_embedding):
    mesh = plsc.VectorSubcoreMesh(
        core_axis_name="c", subcore_axis_name="s", num_cores=NC, num_subcores=NS
    )
    return pl.kernel(
        _sc_body,
        out_type=jax.ShapeDtypeStruct((ROWS, D), jnp.float32),
        mesh=mesh,
        compiler_params=pltpu.CompilerParams(use_tc_tiling_on_sc=False),
        scratch_types=[
            pltpu.VMEM((L + CHUNK, D), jnp.float32),
            pltpu.VMEM((NCHUNK, CHUNK), jnp.int32),
            pltpu.VMEM((NBUF * CHUNK, D), jnp.float32),
        ]
        + [pltpu.SemaphoreType.DMA] * (2 * NBUF),
    )(seq2, pos2, item_embedding)


def kernel(seq, pos, neg, item_embedding, position_embedding):
    del pos, neg
    seq2 = seq.reshape(NW * NCHUNK, CHUNK).astype(jnp.int32)
    pos2 = jnp.concatenate(
        [position_embedding, position_embedding[:CHUNK]], axis=0
    )
    out = _sc_call(seq2, pos2, item_embedding)
    return out.reshape(B, L, D)
